# baseline (speedup 1.0000x reference)
import functools

import jax
import jax.numpy as jnp
from jax import lax
from jax.experimental import pallas as pl
from jax.experimental.pallas import tpu as pltpu
from jax.experimental.pallas import tpu_sc as plsc

N = 16777216
NC = 2
NS = 16
L = 16
NW = NC * NS
PER_W = N // NW
CHUNK = 16384
NCHUNK = PER_W // CHUNK
VECS = CHUNK // L

_mesh = plsc.VectorSubcoreMesh(core_axis_name="c", subcore_axis_name="s")


@functools.partial(
    pl.kernel,
    mesh=_mesh,
    out_type=jax.ShapeDtypeStruct((N,), jnp.float32),
    scratch_types=[
        pltpu.VMEM((CHUNK,), jnp.float32),
        pltpu.VMEM((CHUNK,), jnp.float32),
    ],
)
def _stable_zero_div(x_hbm, y_hbm, out_hbm, x_v, y_v):
    wid = lax.axis_index("s") * NC + lax.axis_index("c")
    base = wid * PER_W

    def chunk_body(ci, carry):
        off = base + ci * CHUNK
        pltpu.sync_copy(x_hbm.at[pl.ds(off, CHUNK)], x_v)
        pltpu.sync_copy(y_hbm.at[pl.ds(off, CHUNK)], y_v)

        def vec_body(vi, c):
            s = pl.ds(vi * L, L)
            xv = x_v[s]
            yv = y_v[s]
            x_v[s] = jnp.where(yv != 0.0, xv / yv, 0.0)
            return c

        lax.fori_loop(0, VECS, vec_body, 0)
        pltpu.sync_copy(x_v, out_hbm.at[pl.ds(off, CHUNK)])
        return carry

    lax.fori_loop(0, NCHUNK, chunk_body, 0)


def kernel(x, y):
    return _stable_zero_div(x, y)


# speedup vs baseline: 1.1548x; 1.1548x over previous
import functools

import jax
import jax.numpy as jnp
from jax import lax
from jax.experimental import pallas as pl
from jax.experimental.pallas import tpu as pltpu
from jax.experimental.pallas import tpu_sc as plsc

N = 16777216
NC = 2
NS = 16
L = 16
NW = NC * NS
PER_W = N // NW
CHUNK = 16384
NCHUNK = PER_W // CHUNK
VECS = CHUNK // L

_mesh = plsc.VectorSubcoreMesh(core_axis_name="c", subcore_axis_name="s")


@functools.partial(
    pl.kernel,
    mesh=_mesh,
    out_type=jax.ShapeDtypeStruct((N,), jnp.float32),
    scratch_types=[
        pltpu.VMEM((2, CHUNK), jnp.float32),
        pltpu.VMEM((2, CHUNK), jnp.float32),
        pltpu.SemaphoreType.DMA,
        pltpu.SemaphoreType.DMA,
        pltpu.SemaphoreType.DMA,
        pltpu.SemaphoreType.DMA,
        pltpu.SemaphoreType.DMA,
        pltpu.SemaphoreType.DMA,
    ],
)
def _stable_zero_div(x_hbm, y_hbm, out_hbm, x_v, y_v,
                     gx0, gx1, gy0, gy1, sc0, sc1):
    sgx = (gx0, gx1)
    sgy = (gy0, gy1)
    ssc = (sc0, sc1)
    wid = lax.axis_index("s") * NC + lax.axis_index("c")
    base = wid * PER_W

    def start_gather(ci, t):
        off = base + ci * CHUNK
        hx = pltpu.async_copy(x_hbm.at[pl.ds(off, CHUNK)], x_v.at[t], sgx[t])
        hy = pltpu.async_copy(y_hbm.at[pl.ds(off, CHUNK)], y_v.at[t], sgy[t])
        return hx, hy

    def start_scatter(ci, t):
        off = base + ci * CHUNK
        return pltpu.async_copy(x_v.at[t], out_hbm.at[pl.ds(off, CHUNK)], ssc[t])

    def compute(t):
        def vec_body(vi, c):
            s = pl.ds(vi * L, L)
            xv = x_v[t, s]
            yv = y_v[t, s]
            x_v[t, s] = jnp.where(yv == 0.0, 0.0, xv / yv)
            return c

        lax.fori_loop(0, VECS, vec_body, 0)

    gat = [None, None]
    sca = [None, None]
    gat[0] = start_gather(0, 0)
    for ci in range(NCHUNK):
        s = ci & 1
        t = 1 - s
        if ci + 1 < NCHUNK:
            if sca[t] is not None:
                sca[t].wait()
            gat[t] = start_gather(ci + 1, t)
        hx, hy = gat[s]
        hx.wait()
        hy.wait()
        compute(s)
        sca[s] = start_scatter(ci, s)
    sca[0].wait()
    sca[1].wait()


def kernel(x, y):
    return _stable_zero_div(x, y)


# speedup vs baseline: 2.2920x; 1.9847x over previous
import functools

import jax
import jax.numpy as jnp
from jax import lax
from jax.experimental import pallas as pl
from jax.experimental.pallas import tpu as pltpu
from jax.experimental.pallas import tpu_sc as plsc

N = 16777216
NC = 2
NS = 16
L = 16
NW = NC * NS
PER_W = N // NW
CHUNK = 16384
NCHUNK = PER_W // CHUNK
VECS = CHUNK // L

_mesh = plsc.VectorSubcoreMesh(core_axis_name="c", subcore_axis_name="s")


@functools.partial(
    pl.kernel,
    mesh=_mesh,
    out_type=jax.ShapeDtypeStruct((N,), jnp.float32),
    scratch_types=[
        pltpu.VMEM((2, CHUNK), jnp.float32),
        pltpu.VMEM((2, CHUNK), jnp.float32),
        pltpu.VMEM((2, CHUNK), jnp.float32),
        pltpu.SemaphoreType.DMA,
        pltpu.SemaphoreType.DMA,
        pltpu.SemaphoreType.DMA,
        pltpu.SemaphoreType.DMA,
        pltpu.SemaphoreType.DMA,
        pltpu.SemaphoreType.DMA,
    ],
)
def _stable_zero_div(x_hbm, y_hbm, out_hbm, x_v, y_v, o_v,
                     gx0, gx1, gy0, gy1, sc0, sc1):
    sgx = (gx0, gx1)
    sgy = (gy0, gy1)
    ssc = (sc0, sc1)
    wid = lax.axis_index("s") * NC + lax.axis_index("c")
    base = wid * PER_W

    def start_gather(ci, t):
        off = base + ci * CHUNK
        hx = pltpu.async_copy(x_hbm.at[pl.ds(off, CHUNK)], x_v.at[t], sgx[t])
        hy = pltpu.async_copy(y_hbm.at[pl.ds(off, CHUNK)], y_v.at[t], sgy[t])
        return hx, hy

    def start_scatter(ci, t):
        off = base + ci * CHUNK
        return pltpu.async_copy(o_v.at[t], out_hbm.at[pl.ds(off, CHUNK)], ssc[t])

    def compute(t):
        @plsc.parallel_loop(0, CHUNK, step=L, unroll=8)
        def vec_body(i):
            s = pl.ds(i, L)
            xv = x_v[t, s]
            yv = y_v[t, s]
            o_v[t, s] = jnp.where(yv == 0.0, 0.0, xv / yv)

    gat = [None, None]
    sca = [None, None]
    gat[0] = start_gather(0, 0)
    for ci in range(NCHUNK):
        s = ci & 1
        t = 1 - s
        if ci + 1 < NCHUNK:
            gat[t] = start_gather(ci + 1, t)
        hx, hy = gat[s]
        hx.wait()
        hy.wait()
        if sca[s] is not None:
            sca[s].wait()
        compute(s)
        sca[s] = start_scatter(ci, s)
    sca[0].wait()
    sca[1].wait()


def kernel(x, y):
    return _stable_zero_div(x, y)
